# baseline (speedup 1.0000x reference)
import functools
import math

import jax
import jax.numpy as jnp
from jax import lax
from jax.experimental import pallas as pl
from jax.experimental.pallas import tpu as pltpu
from jax.experimental.pallas import tpu_sc as plsc

_SIZE = 100000
_PAD = 0
_SV = 0.1 / (_SIZE - 2)
_CONF = 0.9
_C_ROW = (_SIZE - 2) * _SV * math.log(_SV) + _CONF * math.log(_CONF)

_BC = 2048
_GRID = (_SIZE + _BC - 1) // _BC

_N = 1024
_NW = 32
_BPW = _N // _NW


def _sc_gather(flat_hbm, tgt_hbm, g_hbm, idx_v, g_v, sem):
    wid = lax.axis_index("s") * 2 + lax.axis_index("c")
    base = wid * _BPW
    pltpu.sync_copy(tgt_hbm.at[pl.ds(base, _BPW)], idx_v)
    for k in range(_BPW // 16):
        t16 = idx_v[pl.ds(k * 16, 16)]
        rows = base + k * 16 + lax.iota(jnp.int32, 16)
        idx_v[pl.ds(k * 16, 16)] = rows * _SIZE + t16
    pltpu.async_copy(flat_hbm.at[idx_v], g_v, sem).wait()
    pltpu.sync_copy(g_v, g_hbm.at[pl.ds(base, _BPW)])


def _gather_call(flat, tgt):
    mesh = plsc.VectorSubcoreMesh(core_axis_name="c", subcore_axis_name="s")
    return pl.kernel(
        _sc_gather,
        mesh=mesh,
        out_type=jax.ShapeDtypeStruct((_N,), jnp.float32),
        scratch_types=[
            pltpu.VMEM((_BPW,), jnp.int32),
            pltpu.VMEM((_BPW,), jnp.float32),
            pltpu.SemaphoreType.DMA,
        ],
    )(flat, tgt)


def _tc_kernel(mask_ref, out_ref, tgt_ref, acc_ref, rowacc_ref):
    j = pl.program_id(0)
    x = out_ref[...]
    mk = mask_ref[...].reshape(1, _BC)
    xm = jnp.where(mk != 0, x, 0.0)

    @pl.when(j == 0)
    def _init():
        rowacc_ref[...] = jnp.zeros_like(rowacc_ref)

    rowacc_ref[...] += jnp.sum(xm, axis=1, keepdims=True)

    @pl.when(j == _GRID - 1)
    def _combine():
        t = tgt_ref[...]
        per_row = _C_ROW - _SV * rowacc_ref[...]
        acc_ref[0, 0] = jnp.sum(jnp.where(t != _PAD, per_row, 0.0))


def kernel(output, target, normalize):
    tgt = target.astype(jnp.int32).reshape(_N)
    g = _gather_call(output.reshape(-1), tgt)

    cols = jnp.arange(_GRID * _BC, dtype=jnp.int32)
    mask = ((cols != _PAD) & (cols < _SIZE)).astype(jnp.int32)
    mask = mask.reshape(_GRID, 1, _BC)

    acc = pl.pallas_call(
        _tc_kernel,
        grid=(_GRID,),
        in_specs=[
            pl.BlockSpec((1, 1, _BC), lambda j: (j, 0, 0)),
            pl.BlockSpec((_N, _BC), lambda j: (0, j)),
            pl.BlockSpec((_N, 1), lambda j: (0, 0)),
        ],
        out_specs=pl.BlockSpec((1, 1), lambda j: (0, 0), memory_space=pltpu.SMEM),
        out_shape=jax.ShapeDtypeStruct((1, 1), jnp.float32),
        scratch_shapes=[pltpu.VMEM((_N, 1), jnp.float32)],
    )(mask, output, tgt.reshape(_N, 1))
    gather_term = jnp.sum(jnp.where(tgt != _PAD, (_CONF - _SV) * g, 0.0))
    return (acc[0, 0] - gather_term) / jnp.asarray(normalize, dtype=jnp.float32)


# speedup vs baseline: 1.8788x; 1.8788x over previous
import math

import jax
import jax.numpy as jnp
from jax.experimental import pallas as pl
from jax.experimental.pallas import tpu as pltpu

_SIZE = 100000
_PAD = 0
_SV = 0.1 / (_SIZE - 2)
_CONF = 0.9
_C_ROW = (_SIZE - 2) * _SV * math.log(_SV) + _CONF * math.log(_CONF)

_N = 1024
_BM = 8
_GRID = _N // _BM


def _tc_kernel(out_ref, tgt_ref, acc_ref):
    j = pl.program_id(0)
    x = out_ref[...]
    t = tgt_ref[...]
    rowsum = jnp.sum(x, axis=1, keepdims=True) - x[:, 0:1]
    cols = jax.lax.broadcasted_iota(jnp.int32, x.shape, 1)
    g = jnp.sum(jnp.where(cols == t, x, 0.0), axis=1, keepdims=True)
    per_row = _C_ROW - _SV * (rowsum - g) - _CONF * g
    part = jnp.sum(jnp.where(t != _PAD, per_row, 0.0))

    @pl.when(j == 0)
    def _init():
        acc_ref[0, 0] = 0.0

    acc_ref[0, 0] += part


def kernel(output, target, normalize):
    tgt = target.astype(jnp.int32)
    acc = pl.pallas_call(
        _tc_kernel,
        grid=(_GRID,),
        in_specs=[
            pl.BlockSpec((_BM, _SIZE), lambda j: (j, 0)),
            pl.BlockSpec((_BM, 1), lambda j: (j, 0)),
        ],
        out_specs=pl.BlockSpec((1, 1), lambda j: (0, 0), memory_space=pltpu.SMEM),
        out_shape=jax.ShapeDtypeStruct((1, 1), jnp.float32),
    )(output, tgt)
    return acc[0, 0] / jnp.asarray(normalize, dtype=jnp.float32)


# speedup vs baseline: 2.1934x; 1.1674x over previous
import math

import jax
import jax.numpy as jnp
from jax.experimental import pallas as pl
from jax.experimental.pallas import tpu as pltpu

_SIZE = 100000
_PAD = 0
_SV = 0.1 / (_SIZE - 2)
_CONF = 0.9
_C_ROW = (_SIZE - 2) * _SV * math.log(_SV) + _CONF * math.log(_CONF)

_N = 1024
_BC = 4096
_GRID = (_SIZE + _BC - 1) // _BC


def _tc_kernel(mask_ref, out_ref, tgt_ref, acc_ref, rowacc_ref, gacc_ref):
    j = pl.program_id(0)
    x = out_ref[...]
    mk = mask_ref[...].reshape(1, _BC)
    xm = jnp.where(mk != 0, x, 0.0)
    cols = j * _BC + jax.lax.broadcasted_iota(jnp.int32, x.shape, 1)
    t = tgt_ref[...]

    @pl.when(j == 0)
    def _init():
        rowacc_ref[...] = jnp.zeros_like(rowacc_ref)
        gacc_ref[...] = jnp.zeros_like(gacc_ref)

    rowacc_ref[...] += jnp.sum(xm, axis=1, keepdims=True)
    gacc_ref[...] += jnp.sum(jnp.where(cols == t, xm, 0.0), axis=1, keepdims=True)

    @pl.when(j == _GRID - 1)
    def _combine():
        g = gacc_ref[...]
        per_row = _C_ROW - _SV * (rowacc_ref[...] - g) - _CONF * g
        acc_ref[0, 0] = jnp.sum(jnp.where(t != _PAD, per_row, 0.0))


def kernel(output, target, normalize):
    tgt = target.astype(jnp.int32)

    cols = jnp.arange(_GRID * _BC, dtype=jnp.int32)
    mask = ((cols != _PAD) & (cols < _SIZE)).astype(jnp.int32)
    mask = mask.reshape(_GRID, 1, _BC)

    acc = pl.pallas_call(
        _tc_kernel,
        grid=(_GRID,),
        in_specs=[
            pl.BlockSpec((1, 1, _BC), lambda j: (j, 0, 0)),
            pl.BlockSpec((_N, _BC), lambda j: (0, j)),
            pl.BlockSpec((_N, 1), lambda j: (0, 0)),
        ],
        out_specs=pl.BlockSpec((1, 1), lambda j: (0, 0), memory_space=pltpu.SMEM),
        out_shape=jax.ShapeDtypeStruct((1, 1), jnp.float32),
        scratch_shapes=[
            pltpu.VMEM((_N, 1), jnp.float32),
            pltpu.VMEM((_N, 1), jnp.float32),
        ],
    )(mask, output, tgt)
    return acc[0, 0] / jnp.asarray(normalize, dtype=jnp.float32)
